# baseline (speedup 1.0000x reference)
import jax
import jax.numpy as jnp
from jax import lax
from jax.experimental import pallas as pl
from jax.experimental.pallas import tpu as pltpu
from jax.experimental.pallas import tpu_sc as plsc

N = 10000
D = 128
H = 64
C = 16
K = 10
ALPHA = 0.1

NT = 16
RPT = 632
NPAD = NT * RPT
CW = 128
NCH = 157
EPT = NCH * CW
EPAD = NT * EPT

_MESH = plsc.VectorSubcoreMesh(
    core_axis_name="c", subcore_axis_name="s", num_cores=1)
_SC_PARAMS = pltpu.CompilerParams(use_tc_tiling_on_sc=False)


def _rsqrt16(d):
    i = lax.bitcast_convert_type(d, jnp.int32)
    i = 0x5F3759DF - lax.shift_right_arithmetic(i, 1)
    r = lax.bitcast_convert_type(i, jnp.float32)
    for _ in range(3):
        r = r * (1.5 - 0.5 * d * r * r)
    return r


def _graph_body(src_hbm, dst_hbm, h_hbm, yout_hbm, sdeg_hbm,
                s_sp, mysrc, mydst, gbuf, gbuf2, sbuf, ybuf,
                c1t, c2t, sdbuf, sem, sem2):
    t = lax.axis_index("s")
    base = t * RPT
    rows = pl.ds(base, RPT)
    pltpu.sync_copy(src_hbm.at[t], mysrc)
    pltpu.sync_copy(dst_hbm.at[t], mydst)
    pltpu.sync_copy(h_hbm.at[rows], ybuf)

    def zrow(i, c):
        sbuf[i, :] = jnp.zeros((C,), jnp.float32)
        return c
    lax.fori_loop(0, RPT, zrow, 0)
    pltpu.sync_copy(sbuf, s_sp.at[rows])

    def fill_ones(i, c):
        gbuf[i, :] = jnp.ones((C,), jnp.float32)
        return c
    lax.fori_loop(0, CW, fill_ones, 0)
    plsc.subcore_barrier()

    def deg_chunk(j, c):
        pltpu.sync_copy(gbuf, s_sp.at[mydst.at[j]], add=True)
        return c
    lax.fori_loop(0, NCH, deg_chunk, 0)
    plsc.subcore_barrier()

    pltpu.sync_copy(s_sp.at[rows], sbuf)

    def prep_row(i, c):
        d = sbuf[i, :] + 1.0
        dinv = _rsqrt16(d)
        c1t[i, :] = (1.0 - ALPHA) * dinv * dinv
        y0 = dinv * ybuf[i, :]
        ybuf[i, :] = y0
        c2t[i, :] = ALPHA * y0
        sdbuf[i, :] = d * dinv
        sbuf[i, :] = jnp.zeros((C,), jnp.float32)
        return c
    lax.fori_loop(0, RPT, prep_row, 0)
    pltpu.sync_copy(sdbuf, sdeg_hbm.at[rows])
    pltpu.sync_copy(ybuf, yout_hbm.at[rows])
    pltpu.sync_copy(sbuf, s_sp.at[rows])
    plsc.subcore_barrier()

    def round_body(_, carry):
        pltpu.async_copy(yout_hbm.at[mysrc.at[0]], gbuf, sem)

        def pair(i, c):
            j = 2 * i
            pltpu.make_async_copy(yout_hbm.at[mysrc.at[j]], gbuf, sem).wait()
            hb = pltpu.async_copy(yout_hbm.at[mysrc.at[j + 1]], gbuf2, sem2)
            pltpu.sync_copy(gbuf, s_sp.at[mydst.at[j]], add=True)
            pltpu.async_copy(yout_hbm.at[mysrc.at[j + 2]], gbuf, sem)
            hb.wait()
            pltpu.sync_copy(gbuf2, s_sp.at[mydst.at[j + 1]], add=True)
            return c
        lax.fori_loop(0, (NCH - 1) // 2, pair, 0)
        pltpu.make_async_copy(yout_hbm.at[mysrc.at[NCH - 1]], gbuf, sem).wait()
        pltpu.sync_copy(gbuf, s_sp.at[mydst.at[NCH - 1]], add=True)
        plsc.subcore_barrier()
        pltpu.sync_copy(s_sp.at[rows], sbuf)

        def row(i, c):
            ybuf[i, :] = c1t[i, :] * (sbuf[i, :] + ybuf[i, :]) + c2t[i, :]
            sbuf[i, :] = jnp.zeros((C,), jnp.float32)
            return c
        lax.fori_loop(0, RPT, row, 0)
        pltpu.sync_copy(ybuf, yout_hbm.at[rows])
        pltpu.sync_copy(sbuf, s_sp.at[rows])
        plsc.subcore_barrier()
        return carry
    lax.fori_loop(0, K, round_body, 0)


_graph_call = pl.kernel(
    _graph_body,
    out_type=(
        jax.ShapeDtypeStruct((NPAD, C), jnp.float32),
        jax.ShapeDtypeStruct((NPAD, C), jnp.float32),
    ),
    mesh=_MESH,
    scratch_types=[
        pltpu.VMEM_SHARED((NPAD, C), jnp.float32),
        pltpu.VMEM((NCH, CW), jnp.int32),
        pltpu.VMEM((NCH, CW), jnp.int32),
        pltpu.VMEM((CW, C), jnp.float32),
        pltpu.VMEM((CW, C), jnp.float32),
        pltpu.VMEM((RPT, C), jnp.float32),
        pltpu.VMEM((RPT, C), jnp.float32),
        pltpu.VMEM((RPT, C), jnp.float32),
        pltpu.VMEM((RPT, C), jnp.float32),
        pltpu.VMEM((RPT, C), jnp.float32),
        pltpu.SemaphoreType.DMA,
        pltpu.SemaphoreType.DMA,
    ],
    compiler_params=_SC_PARAMS,
)


def _mlp_body(x_ref, w1_ref, b1_ref, w2_ref, b2_ref, h_ref):
    h1 = jnp.maximum(
        jnp.dot(x_ref[...], w1_ref[...], preferred_element_type=jnp.float32)
        + b1_ref[...], 0.0)
    h_ref[...] = (
        jnp.dot(h1, w2_ref[...], preferred_element_type=jnp.float32)
        + b2_ref[...])


_mlp_call = pl.pallas_call(
    _mlp_body,
    out_shape=jax.ShapeDtypeStruct((NPAD, C), jnp.float32),
)


def _lsm_body(y_ref, sdeg_ref, out_ref):
    z = y_ref[...] * sdeg_ref[...]
    m = jnp.max(z, axis=1, keepdims=True)
    e = jnp.exp(z - m)
    out_ref[...] = z - m - jnp.log(jnp.sum(e, axis=1, keepdims=True))


_lsm_call = pl.pallas_call(
    _lsm_body,
    out_shape=jax.ShapeDtypeStruct((NPAD, C), jnp.float32),
)


def kernel(x, edge_index, W1, b1, W2, b2):
    src = edge_index[0].astype(jnp.int32)
    dst = edge_index[1].astype(jnp.int32)
    padv = jnp.full((EPAD - src.shape[0],), N, jnp.int32)
    src3 = jnp.concatenate([src, padv]).reshape(NT, NCH, CW)
    dst3 = jnp.concatenate([dst, padv]).reshape(NT, NCH, CW)
    xp = jnp.pad(x, ((0, NPAD - N), (0, 0)))

    h = _mlp_call(xp, W1, b1.reshape(1, H), W2, b2.reshape(1, C))
    y, sdeg = _graph_call(src3, dst3, h)
    out = _lsm_call(y, sdeg)
    return out[:N]


# speedup vs baseline: 1.5706x; 1.5706x over previous
import jax
import jax.numpy as jnp
from jax import lax
from jax.experimental import pallas as pl
from jax.experimental.pallas import tpu as pltpu
from jax.experimental.pallas import tpu_sc as plsc

N = 10000
D = 128
H = 64
C = 16
K = 10
ALPHA = 0.1

NT = 16
RPT = 632
NPAD = NT * RPT
CW = 128
NCH = 160
EPT = NCH * CW
EPAD = NT * EPT

_MESH = plsc.VectorSubcoreMesh(
    core_axis_name="c", subcore_axis_name="s", num_cores=1)
_SC_PARAMS = pltpu.CompilerParams(use_tc_tiling_on_sc=False)


def _rsqrt16(d):
    i = lax.bitcast_convert_type(d, jnp.int32)
    i = 0x5F3759DF - lax.shift_right_arithmetic(i, 1)
    r = lax.bitcast_convert_type(i, jnp.float32)
    for _ in range(3):
        r = r * (1.5 - 0.5 * d * r * r)
    return r


def _graph_body(src_hbm, dst_hbm, h_hbm, yout_hbm, sdeg_hbm,
                y_sp, s_sp, mysrc, mydst, g0, g1, g2, g3, sbuf, ybuf,
                c1t, c2t, sdbuf, sg0, sg1, sg2, sg3, ss0, ss1, ss2, ss3):
    gbufs = (g0, g1, g2, g3)
    semg = (sg0, sg1, sg2, sg3)
    sems = (ss0, ss1, ss2, ss3)
    t = lax.axis_index("s")
    base = t * RPT
    rows = pl.ds(base, RPT)
    pltpu.sync_copy(src_hbm.at[t], mysrc)
    pltpu.sync_copy(dst_hbm.at[t], mydst)
    pltpu.sync_copy(h_hbm.at[rows], ybuf)

    def zrow(i, c):
        sbuf[i, :] = jnp.zeros((C,), jnp.float32)
        return c
    lax.fori_loop(0, RPT, zrow, 0)
    pltpu.sync_copy(sbuf, s_sp.at[rows])

    def fill_ones(i, c):
        g0[i, :] = jnp.ones((C,), jnp.float32)
        return c
    lax.fori_loop(0, CW, fill_ones, 0)
    plsc.subcore_barrier()

    def deg_chunk(j, c):
        pltpu.sync_copy(g0, s_sp.at[mydst.at[j]], add=True)
        return c
    lax.fori_loop(0, NCH, deg_chunk, 0)
    plsc.subcore_barrier()

    pltpu.sync_copy(s_sp.at[rows], sbuf)

    def prep_row(i, c):
        d = sbuf[i, :] + 1.0
        dinv = _rsqrt16(d)
        c1t[i, :] = (1.0 - ALPHA) * dinv * dinv
        y0 = dinv * ybuf[i, :]
        ybuf[i, :] = y0
        c2t[i, :] = ALPHA * y0
        sdbuf[i, :] = d * dinv
        sbuf[i, :] = jnp.zeros((C,), jnp.float32)
        return c
    lax.fori_loop(0, RPT, prep_row, 0)
    pltpu.sync_copy(sdbuf, sdeg_hbm.at[rows])
    pltpu.sync_copy(ybuf, y_sp.at[rows])
    pltpu.sync_copy(sbuf, s_sp.at[rows])
    plsc.subcore_barrier()

    def round_body(_, carry):
        for b in range(4):
            pltpu.async_copy(y_sp.at[mysrc.at[b]], gbufs[b], semg[b])

        def group(i, c):
            j0 = 4 * i
            hs = []
            for b in range(4):
                pltpu.make_async_copy(
                    y_sp.at[mysrc.at[j0 + b]], gbufs[b], semg[b]).wait()
                hs.append(pltpu.async_copy(
                    gbufs[b], s_sp.at[mydst.at[j0 + b]], sems[b], add=True))
            for b in range(4):
                hs[b].wait()
                pltpu.async_copy(
                    y_sp.at[mysrc.at[j0 + 4 + b]], gbufs[b], semg[b])
            return c
        lax.fori_loop(0, NCH // 4 - 1, group, 0)
        hs = []
        for b in range(4):
            jj = NCH - 4 + b
            pltpu.make_async_copy(
                y_sp.at[mysrc.at[jj]], gbufs[b], semg[b]).wait()
            hs.append(pltpu.async_copy(
                gbufs[b], s_sp.at[mydst.at[jj]], sems[b], add=True))
        for b in range(4):
            hs[b].wait()
        plsc.subcore_barrier()
        pltpu.sync_copy(s_sp.at[rows], sbuf)

        def row(i, c):
            ybuf[i, :] = c1t[i, :] * (sbuf[i, :] + ybuf[i, :]) + c2t[i, :]
            sbuf[i, :] = jnp.zeros((C,), jnp.float32)
            return c
        lax.fori_loop(0, RPT, row, 0)
        pltpu.sync_copy(ybuf, y_sp.at[rows])
        pltpu.sync_copy(sbuf, s_sp.at[rows])
        plsc.subcore_barrier()
        return carry
    lax.fori_loop(0, K, round_body, 0)
    pltpu.sync_copy(ybuf, yout_hbm.at[rows])


_graph_call = pl.kernel(
    _graph_body,
    out_type=(
        jax.ShapeDtypeStruct((NPAD, C), jnp.float32),
        jax.ShapeDtypeStruct((NPAD, C), jnp.float32),
    ),
    mesh=_MESH,
    scratch_types=[
        pltpu.VMEM_SHARED((NPAD, C), jnp.float32),
        pltpu.VMEM_SHARED((NPAD, C), jnp.float32),
        pltpu.VMEM((NCH, CW), jnp.int32),
        pltpu.VMEM((NCH, CW), jnp.int32),
        pltpu.VMEM((CW, C), jnp.float32),
        pltpu.VMEM((CW, C), jnp.float32),
        pltpu.VMEM((CW, C), jnp.float32),
        pltpu.VMEM((CW, C), jnp.float32),
        pltpu.VMEM((RPT, C), jnp.float32),
        pltpu.VMEM((RPT, C), jnp.float32),
        pltpu.VMEM((RPT, C), jnp.float32),
        pltpu.VMEM((RPT, C), jnp.float32),
        pltpu.VMEM((RPT, C), jnp.float32),
        pltpu.SemaphoreType.DMA,
        pltpu.SemaphoreType.DMA,
        pltpu.SemaphoreType.DMA,
        pltpu.SemaphoreType.DMA,
        pltpu.SemaphoreType.DMA,
        pltpu.SemaphoreType.DMA,
        pltpu.SemaphoreType.DMA,
        pltpu.SemaphoreType.DMA,
    ],
    compiler_params=_SC_PARAMS,
)


def _mlp_body(x_ref, w1_ref, b1_ref, w2_ref, b2_ref, h_ref):
    h1 = jnp.maximum(
        jnp.dot(x_ref[...], w1_ref[...], preferred_element_type=jnp.float32)
        + b1_ref[...], 0.0)
    h_ref[...] = (
        jnp.dot(h1, w2_ref[...], preferred_element_type=jnp.float32)
        + b2_ref[...])


_mlp_call = pl.pallas_call(
    _mlp_body,
    out_shape=jax.ShapeDtypeStruct((NPAD, C), jnp.float32),
)


def _lsm_body(y_ref, sdeg_ref, out_ref):
    z = y_ref[...] * sdeg_ref[...]
    m = jnp.max(z, axis=1, keepdims=True)
    e = jnp.exp(z - m)
    out_ref[...] = z - m - jnp.log(jnp.sum(e, axis=1, keepdims=True))


_lsm_call = pl.pallas_call(
    _lsm_body,
    out_shape=jax.ShapeDtypeStruct((NPAD, C), jnp.float32),
)


def kernel(x, edge_index, W1, b1, W2, b2):
    src = edge_index[0].astype(jnp.int32)
    dst = edge_index[1].astype(jnp.int32)
    padv = jnp.full((EPAD - src.shape[0],), N, jnp.int32)
    src3 = jnp.concatenate([src, padv]).reshape(NT, NCH, CW)
    dst3 = jnp.concatenate([dst, padv]).reshape(NT, NCH, CW)
    xp = jnp.pad(x, ((0, NPAD - N), (0, 0)))

    h = _mlp_call(xp, W1, b1.reshape(1, H), W2, b2.reshape(1, C))
    y, sdeg = _graph_call(src3, dst3, h)
    out = _lsm_call(y, sdeg)
    return out[:N]


# speedup vs baseline: 1.8386x; 1.1706x over previous
import jax
import jax.numpy as jnp
from jax import lax
from jax.experimental import pallas as pl
from jax.experimental.pallas import tpu as pltpu
from jax.experimental.pallas import tpu_sc as plsc

N = 10000
D = 128
H = 64
C = 16
K = 10
ALPHA = 0.1

NT = 16
RPT = 632
NPAD = NT * RPT
CW = 128
NCH = 157
EPT = NCH * CW
EPAD = NT * EPT

_MESH = plsc.VectorSubcoreMesh(
    core_axis_name="c", subcore_axis_name="s", num_cores=1)
_SC_PARAMS = pltpu.CompilerParams(use_tc_tiling_on_sc=False)


def _rsqrt16(d):
    i = lax.bitcast_convert_type(d, jnp.int32)
    i = 0x5F3759DF - lax.shift_right_arithmetic(i, 1)
    r = lax.bitcast_convert_type(i, jnp.float32)
    for _ in range(3):
        r = r * (1.5 - 0.5 * d * r * r)
    return r


def _graph_body(src_hbm, dst_hbm, h_hbm, yout_hbm, sdeg_hbm,
                y_sp, s_sp, mysrc, mydst, gbuf, gbuf2, sbuf, ybuf,
                c1t, c2t, sdbuf, sem, sem2):
    t = lax.axis_index("s")
    base = t * RPT
    rows = pl.ds(base, RPT)
    pltpu.sync_copy(src_hbm.at[t], mysrc)
    pltpu.sync_copy(dst_hbm.at[t], mydst)
    pltpu.sync_copy(h_hbm.at[rows], ybuf)

    def zrow(i, c):
        sbuf[i, :] = jnp.zeros((C,), jnp.float32)
        return c
    lax.fori_loop(0, RPT, zrow, 0)
    pltpu.sync_copy(sbuf, s_sp.at[rows])

    def fill_ones(i, c):
        gbuf[i, :] = jnp.ones((C,), jnp.float32)
        return c
    lax.fori_loop(0, CW, fill_ones, 0)
    plsc.subcore_barrier()

    def deg_chunk(j, c):
        pltpu.sync_copy(gbuf, s_sp.at[mydst.at[j]], add=True)
        return c
    lax.fori_loop(0, NCH, deg_chunk, 0)
    plsc.subcore_barrier()

    pltpu.sync_copy(s_sp.at[rows], sbuf)

    def prep_row(i, c):
        d = sbuf[i, :] + 1.0
        dinv = _rsqrt16(d)
        c1t[i, :] = (1.0 - ALPHA) * dinv * dinv
        y0 = dinv * ybuf[i, :]
        ybuf[i, :] = y0
        c2t[i, :] = ALPHA * y0
        sdbuf[i, :] = d * dinv
        sbuf[i, :] = jnp.zeros((C,), jnp.float32)
        return c
    lax.fori_loop(0, RPT, prep_row, 0)
    pltpu.sync_copy(sdbuf, sdeg_hbm.at[rows])
    pltpu.sync_copy(ybuf, y_sp.at[rows])
    pltpu.sync_copy(sbuf, s_sp.at[rows])
    plsc.subcore_barrier()

    def round_body(_, carry):
        pltpu.async_copy(y_sp.at[mysrc.at[0]], gbuf, sem)

        def pair(i, c):
            j = 2 * i
            pltpu.make_async_copy(y_sp.at[mysrc.at[j]], gbuf, sem).wait()
            hb = pltpu.async_copy(y_sp.at[mysrc.at[j + 1]], gbuf2, sem2)
            pltpu.sync_copy(gbuf, s_sp.at[mydst.at[j]], add=True)
            pltpu.async_copy(y_sp.at[mysrc.at[j + 2]], gbuf, sem)
            hb.wait()
            pltpu.sync_copy(gbuf2, s_sp.at[mydst.at[j + 1]], add=True)
            return c
        lax.fori_loop(0, (NCH - 1) // 2, pair, 0)
        pltpu.make_async_copy(y_sp.at[mysrc.at[NCH - 1]], gbuf, sem).wait()
        pltpu.sync_copy(gbuf, s_sp.at[mydst.at[NCH - 1]], add=True)
        plsc.subcore_barrier()
        pltpu.sync_copy(s_sp.at[rows], sbuf)

        def row(i, c):
            ybuf[i, :] = c1t[i, :] * (sbuf[i, :] + ybuf[i, :]) + c2t[i, :]
            sbuf[i, :] = jnp.zeros((C,), jnp.float32)
            return c
        lax.fori_loop(0, RPT, row, 0)
        pltpu.sync_copy(ybuf, y_sp.at[rows])
        pltpu.sync_copy(sbuf, s_sp.at[rows])
        plsc.subcore_barrier()
        return carry
    lax.fori_loop(0, K, round_body, 0)
    pltpu.sync_copy(ybuf, yout_hbm.at[rows])


_graph_call = pl.kernel(
    _graph_body,
    out_type=(
        jax.ShapeDtypeStruct((NPAD, C), jnp.float32),
        jax.ShapeDtypeStruct((NPAD, C), jnp.float32),
    ),
    mesh=_MESH,
    scratch_types=[
        pltpu.VMEM_SHARED((NPAD, C), jnp.float32),
        pltpu.VMEM_SHARED((NPAD, C), jnp.float32),
        pltpu.VMEM((NCH, CW), jnp.int32),
        pltpu.VMEM((NCH, CW), jnp.int32),
        pltpu.VMEM((CW, C), jnp.float32),
        pltpu.VMEM((CW, C), jnp.float32),
        pltpu.VMEM((RPT, C), jnp.float32),
        pltpu.VMEM((RPT, C), jnp.float32),
        pltpu.VMEM((RPT, C), jnp.float32),
        pltpu.VMEM((RPT, C), jnp.float32),
        pltpu.VMEM((RPT, C), jnp.float32),
        pltpu.SemaphoreType.DMA,
        pltpu.SemaphoreType.DMA,
    ],
    compiler_params=_SC_PARAMS,
)


def _mlp_body(x_ref, w1_ref, b1_ref, w2_ref, b2_ref, h_ref):
    h1 = jnp.maximum(
        jnp.dot(x_ref[...], w1_ref[...], preferred_element_type=jnp.float32)
        + b1_ref[...], 0.0)
    h_ref[...] = (
        jnp.dot(h1, w2_ref[...], preferred_element_type=jnp.float32)
        + b2_ref[...])


_mlp_call = pl.pallas_call(
    _mlp_body,
    out_shape=jax.ShapeDtypeStruct((NPAD, C), jnp.float32),
)


def _lsm_body(y_ref, sdeg_ref, out_ref):
    z = y_ref[...] * sdeg_ref[...]
    m = jnp.max(z, axis=1, keepdims=True)
    e = jnp.exp(z - m)
    out_ref[...] = z - m - jnp.log(jnp.sum(e, axis=1, keepdims=True))


_lsm_call = pl.pallas_call(
    _lsm_body,
    out_shape=jax.ShapeDtypeStruct((NPAD, C), jnp.float32),
)


def kernel(x, edge_index, W1, b1, W2, b2):
    src = edge_index[0].astype(jnp.int32)
    dst = edge_index[1].astype(jnp.int32)
    padv = jnp.full((EPAD - src.shape[0],), N, jnp.int32)
    src3 = jnp.concatenate([src, padv]).reshape(NT, NCH, CW)
    dst3 = jnp.concatenate([dst, padv]).reshape(NT, NCH, CW)
    xp = jnp.pad(x, ((0, NPAD - N), (0, 0)))

    h = _mlp_call(xp, W1, b1.reshape(1, H), W2, b2.reshape(1, C))
    y, sdeg = _graph_call(src3, dst3, h)
    out = _lsm_call(y, sdeg)
    return out[:N]
